# baseline (speedup 1.0000x reference)
import functools

import jax
import jax.numpy as jnp
from jax import lax
from jax.experimental import pallas as pl
from jax.experimental.pallas import tpu as pltpu
from jax.experimental.pallas import tpu_sc as plsc

EMB = 128
MAX_DEPTH = 20
NTYPE = 98
NATTR = 10030
N = 100000
C = 80
K = N // C
NC = 2
NS = 16
NW = NC * NS
LANES = 16
CPW = K // NW
MAXCH = CPW + 1
NBUF = 3


def _prelude_body(t_ref, d_ref, x_ref, td_ref, x0_ref, x1_ref):
    td_ref[...] = t_ref[...][:, None, :] + d_ref[...][None, :, :]
    x0_ref[...] = x_ref[:, 0]
    x1_ref[...] = x_ref[:, 1]


def _encoder(td_hbm, atab_hbm, x0_hbm, x1_hbm, dep_hbm, out_hbm,
             x0_v, x1_v, dep_v, comb_v,
             td0, td1, td2, a0, a1, a2, s0, s1, s2,
             isem, g0, g1, g2, o0, o1, o2):
    cc = lax.axis_index("c")
    ss = lax.axis_index("s")
    w = ss * NC + cc
    start = CPW * w + jnp.minimum(w, K % NW)
    n = CPW + jnp.where(w < K % NW, 1, 0)

    rbase = pl.multiple_of(start * C, 8)
    pltpu.async_copy(x0_hbm.at[pl.ds(rbase, CPW * C)], x0_v.at[pl.ds(0, CPW * C)], isem)
    pltpu.async_copy(x1_hbm.at[pl.ds(rbase, CPW * C)], x1_v.at[pl.ds(0, CPW * C)], isem)
    pltpu.async_copy(dep_hbm.at[pl.ds(rbase, CPW * C)], dep_v.at[pl.ds(0, CPW * C)], isem)
    pltpu.make_async_copy(x0_hbm.at[pl.ds(0, CPW * C)], x0_v.at[pl.ds(0, CPW * C)], isem).wait()
    pltpu.make_async_copy(x1_hbm.at[pl.ds(0, CPW * C)], x1_v.at[pl.ds(0, CPW * C)], isem).wait()
    pltpu.make_async_copy(dep_hbm.at[pl.ds(0, CPW * C)], dep_v.at[pl.ds(0, CPW * C)], isem).wait()

    @pl.when(w < K % NW)
    def _():
        ebase = pl.multiple_of((start + CPW) * C, 8)
        pltpu.sync_copy(x0_hbm.at[pl.ds(ebase, C)], x0_v.at[pl.ds(CPW * C, C)])
        pltpu.sync_copy(x1_hbm.at[pl.ds(ebase, C)], x1_v.at[pl.ds(CPW * C, C)])
        pltpu.sync_copy(dep_hbm.at[pl.ds(ebase, C)], dep_v.at[pl.ds(CPW * C, C)])

    @plsc.parallel_loop(0, MAXCH * C // LANES, unroll=4)
    def _(i):
        sl = pl.ds(i * LANES, LANES)
        t = jnp.clip(x0_v[sl], 0, NTYPE - 1)
        d = jnp.clip(dep_v[sl], 0, MAX_DEPTH)
        comb_v[sl] = t * (MAX_DEPTH + 1) + d
        x1_v[sl] = jnp.clip(x1_v[sl], 0, NATTR - 1)

    slots = ((td0, a0, s0, g0, o0), (td1, a1, s1, g1, o1), (td2, a2, s2, g2, o2))

    def fire(jj, tdb, ab, gsem):
        pltpu.async_copy(td_hbm.at[comb_v.at[pl.ds(jj * C, C)]], tdb, gsem)
        pltpu.async_copy(atab_hbm.at[x1_v.at[pl.ds(jj * C, C)]], ab, gsem)

    for b in range(NBUF):
        fire(b, slots[b][0], slots[b][1], slots[b][3])

    def step(jt, carry):
        for b in range(NBUF):
            tdb, ab, sb, gsem, osem = slots[b]
            jj = NBUF * jt + b

            @pl.when(jj < n)
            def _():
                pltpu.make_async_copy(td_hbm.at[comb_v.at[pl.ds(jj * C, C)]], tdb, gsem).wait()
                pltpu.make_async_copy(atab_hbm.at[x1_v.at[pl.ds(jj * C, C)]], ab, gsem).wait()

                @pl.when(jt > 0)
                def _():
                    pltpu.make_async_copy(sb, out_hbm.at[pl.ds(0, C)], osem).wait()

                @plsc.parallel_loop(0, C, unroll=4)
                def _(r):
                    for k in range(EMB // LANES):
                        sl = pl.ds(k * LANES, LANES)
                        sb[r, sl] = tdb[r, sl] + ab[r, sl]

                base = pl.multiple_of((start + jj) * C, 8)
                pltpu.async_copy(sb, out_hbm.at[pl.ds(base, C)], osem)

                @pl.when(jj + NBUF < n)
                def _():
                    fire(jj + NBUF, tdb, ab, gsem)

        return carry

    lax.fori_loop(0, (MAXCH + NBUF - 1) // NBUF, step, 0)
    pltpu.make_async_copy(s0, out_hbm.at[pl.ds(0, C)], o0).wait()
    pltpu.make_async_copy(s1, out_hbm.at[pl.ds(0, C)], o1).wait()
    pltpu.make_async_copy(s2, out_hbm.at[pl.ds(0, C)], o2).wait()


@jax.jit
def _run(x, depth, type_table, attr_table, depth_table):
    td3, x0, x1 = pl.pallas_call(
        _prelude_body,
        out_shape=(
            jax.ShapeDtypeStruct((NTYPE, MAX_DEPTH + 1, EMB), jnp.float32),
            jax.ShapeDtypeStruct((N,), jnp.int32),
            jax.ShapeDtypeStruct((N,), jnp.int32),
        ),
    )(type_table, depth_table, x)
    td = td3.reshape(NTYPE * (MAX_DEPTH + 1), EMB)

    enc = functools.partial(
        pl.kernel,
        mesh=plsc.VectorSubcoreMesh(core_axis_name="c", subcore_axis_name="s"),
        out_type=jax.ShapeDtypeStruct((N, EMB), jnp.float32),
        compiler_params=pltpu.CompilerParams(needs_layout_passes=False),
        scratch_types=[
            pltpu.VMEM((MAXCH * C,), jnp.int32),
            pltpu.VMEM((MAXCH * C,), jnp.int32),
            pltpu.VMEM((MAXCH * C,), jnp.int32),
            pltpu.VMEM((MAXCH * C,), jnp.int32),
            pltpu.VMEM((C, EMB), jnp.float32),
            pltpu.VMEM((C, EMB), jnp.float32),
            pltpu.VMEM((C, EMB), jnp.float32),
            pltpu.VMEM((C, EMB), jnp.float32),
            pltpu.VMEM((C, EMB), jnp.float32),
            pltpu.VMEM((C, EMB), jnp.float32),
            pltpu.VMEM((C, EMB), jnp.float32),
            pltpu.VMEM((C, EMB), jnp.float32),
            pltpu.VMEM((C, EMB), jnp.float32),
            pltpu.SemaphoreType.DMA,
            pltpu.SemaphoreType.DMA,
            pltpu.SemaphoreType.DMA,
            pltpu.SemaphoreType.DMA,
            pltpu.SemaphoreType.DMA,
            pltpu.SemaphoreType.DMA,
            pltpu.SemaphoreType.DMA,
        ],
    )(_encoder)
    return enc(td, attr_table, x0, x1, depth)


def kernel(x, depth, type_table, attr_table, depth_table):
    return _run(x, depth, type_table, attr_table, depth_table)


# speedup vs baseline: 1.6603x; 1.6603x over previous
import functools

import jax
import jax.numpy as jnp
from jax import lax
from jax.experimental import pallas as pl
from jax.experimental.pallas import tpu as pltpu
from jax.experimental.pallas import tpu_sc as plsc

EMB = 128
MAX_DEPTH = 20
NTYPE = 98
NATTR = 10030
N = 100000
U = 80
K = N // U
NC = 2
NS = 16
NW = NC * NS
LANES = 16
UPW = K // NW
MAXU = UPW + 1
C = 128
NFULL = UPW * U // C
TAIL = UPW * U - NFULL * C


def _td_fuse_body(t_ref, d_ref, o_ref):
    o_ref[...] = t_ref[...][:, None, :] + d_ref[...][None, :, :]


def _encoder(td_hbm, atab_hbm, x0_hbm, x1_hbm, dep_hbm, out_hbm,
             x0_v, x1_v, dep_v, comb_v,
             td0, td1, a0, a1, s0, s1,
             isem, g0, g1, o0, o1):
    cc = lax.axis_index("c")
    ss = lax.axis_index("s")
    w = ss * NC + cc
    start = UPW * w + jnp.minimum(w, K % NW)
    rb = pl.multiple_of(start * U, 8)
    extra = w < K % NW
    nfull = NFULL + jnp.where(extra, 1, 0)

    pltpu.async_copy(x0_hbm.at[pl.ds(rb, UPW * U)], x0_v.at[pl.ds(0, UPW * U)], isem)
    pltpu.async_copy(x1_hbm.at[pl.ds(rb, UPW * U)], x1_v.at[pl.ds(0, UPW * U)], isem)
    pltpu.async_copy(dep_hbm.at[pl.ds(rb, UPW * U)], dep_v.at[pl.ds(0, UPW * U)], isem)
    pltpu.make_async_copy(x0_hbm.at[pl.ds(0, UPW * U)], x0_v.at[pl.ds(0, UPW * U)], isem).wait()
    pltpu.make_async_copy(x1_hbm.at[pl.ds(0, UPW * U)], x1_v.at[pl.ds(0, UPW * U)], isem).wait()
    pltpu.make_async_copy(dep_hbm.at[pl.ds(0, UPW * U)], dep_v.at[pl.ds(0, UPW * U)], isem).wait()

    @pl.when(extra)
    def _():
        ebase = pl.multiple_of((start + UPW) * U, 8)
        pltpu.sync_copy(x0_hbm.at[pl.ds(ebase, U)], x0_v.at[pl.ds(UPW * U, U)])
        pltpu.sync_copy(x1_hbm.at[pl.ds(ebase, U)], x1_v.at[pl.ds(UPW * U, U)])
        pltpu.sync_copy(dep_hbm.at[pl.ds(ebase, U)], dep_v.at[pl.ds(UPW * U, U)])

    @plsc.parallel_loop(0, MAXU * U // LANES, unroll=4)
    def _(i):
        sl = pl.ds(i * LANES, LANES)
        t = jnp.clip(x0_v[sl], 0, NTYPE - 1)
        d = jnp.clip(dep_v[sl], 0, MAX_DEPTH)
        comb_v[sl] = t * (MAX_DEPTH + 1) + d
        x1_v[sl] = jnp.clip(x1_v[sl], 0, NATTR - 1)

    slots = ((td0, a0, s0, g0, o0), (td1, a1, s1, g1, o1))

    def fire(jj, tdb, ab, gsem):
        pltpu.async_copy(td_hbm.at[comb_v.at[pl.ds(jj * C, C)]], tdb, gsem)
        pltpu.async_copy(atab_hbm.at[x1_v.at[pl.ds(jj * C, C)]], ab, gsem)

    fire(0, td0, a0, g0)
    fire(1, td1, a1, g1)

    def pair(jp, carry):
        for b in range(2):
            tdb, ab, sb, gsem, osem = slots[b]
            jj = 2 * jp + b

            @pl.when(jj < nfull)
            def _():
                pltpu.make_async_copy(td_hbm.at[comb_v.at[pl.ds(jj * C, C)]], tdb, gsem).wait()
                pltpu.make_async_copy(atab_hbm.at[x1_v.at[pl.ds(jj * C, C)]], ab, gsem).wait()

                @pl.when(jp > 0)
                def _():
                    pltpu.make_async_copy(sb, out_hbm.at[pl.ds(0, C * EMB)], osem).wait()

                @plsc.parallel_loop(0, C, unroll=4)
                def _(r):
                    for k in range(EMB // LANES):
                        sl = pl.ds(k * LANES, LANES)
                        sb[pl.ds(r * EMB + k * LANES, LANES)] = tdb[r, sl] + ab[r, sl]

                obase = pl.multiple_of(rb * EMB + jj * (C * EMB), 8)
                pltpu.async_copy(sb, out_hbm.at[pl.ds(obase, C * EMB)], osem)

                @pl.when(jj + 2 < nfull)
                def _():
                    fire(jj + 2, tdb, ab, gsem)

        return carry

    lax.fori_loop(0, (NFULL + 2) // 2, pair, 0)
    pltpu.make_async_copy(s0, out_hbm.at[pl.ds(0, C * EMB)], o0).wait()
    pltpu.make_async_copy(s1, out_hbm.at[pl.ds(0, C * EMB)], o1).wait()

    @pl.when(jnp.logical_not(extra))
    def _():
        tb = NFULL * C
        pltpu.async_copy(td_hbm.at[comb_v.at[pl.ds(tb, TAIL)]], td0.at[pl.ds(0, TAIL)], g0)
        pltpu.async_copy(atab_hbm.at[x1_v.at[pl.ds(tb, TAIL)]], a0.at[pl.ds(0, TAIL)], g0)
        pltpu.make_async_copy(td_hbm.at[comb_v.at[pl.ds(tb, TAIL)]], td0.at[pl.ds(0, TAIL)], g0).wait()
        pltpu.make_async_copy(atab_hbm.at[x1_v.at[pl.ds(tb, TAIL)]], a0.at[pl.ds(0, TAIL)], g0).wait()

        @plsc.parallel_loop(0, TAIL, unroll=4)
        def _(r):
            for k in range(EMB // LANES):
                sl = pl.ds(k * LANES, LANES)
                s0[pl.ds(r * EMB + k * LANES, LANES)] = td0[r, sl] + a0[r, sl]

        obase = pl.multiple_of(rb * EMB + tb * EMB, 8)
        pltpu.sync_copy(s0.at[pl.ds(0, TAIL * EMB)], out_hbm.at[pl.ds(obase, TAIL * EMB)])


@jax.jit
def _run(x0, x1, depth, type_table, attr_table, depth_table):
    td3 = pl.pallas_call(
        _td_fuse_body,
        out_shape=jax.ShapeDtypeStruct((NTYPE, MAX_DEPTH + 1, EMB), jnp.float32),
    )(type_table, depth_table)
    td = td3.reshape(NTYPE * (MAX_DEPTH + 1), EMB)

    enc = functools.partial(
        pl.kernel,
        mesh=plsc.VectorSubcoreMesh(core_axis_name="c", subcore_axis_name="s"),
        out_type=jax.ShapeDtypeStruct((N * EMB,), jnp.float32),
        compiler_params=pltpu.CompilerParams(needs_layout_passes=False),
        scratch_types=[
            pltpu.VMEM((MAXU * U,), jnp.int32),
            pltpu.VMEM((MAXU * U,), jnp.int32),
            pltpu.VMEM((MAXU * U,), jnp.int32),
            pltpu.VMEM((MAXU * U,), jnp.int32),
            pltpu.VMEM((C, EMB), jnp.float32),
            pltpu.VMEM((C, EMB), jnp.float32),
            pltpu.VMEM((C, EMB), jnp.float32),
            pltpu.VMEM((C, EMB), jnp.float32),
            pltpu.VMEM((C * EMB,), jnp.float32),
            pltpu.VMEM((C * EMB,), jnp.float32),
            pltpu.SemaphoreType.DMA,
            pltpu.SemaphoreType.DMA,
            pltpu.SemaphoreType.DMA,
            pltpu.SemaphoreType.DMA,
            pltpu.SemaphoreType.DMA,
        ],
    )(_encoder)
    flat = enc(td, attr_table, x0, x1, depth)
    return flat.reshape(N, EMB)


def kernel(x, depth, type_table, attr_table, depth_table):
    return _run(x[:, 0], x[:, 1], depth, type_table, attr_table, depth_table)
